# baseline (speedup 1.0000x reference)
import dataclasses
import functools

import jax
import jax.numpy as jnp
from jax import lax
from jax.experimental import pallas as pl
from jax.experimental.pallas import tpu as pltpu
from jax.experimental.pallas import tpu_sc as plsc

_B, _W, _H, _L = 4, 32, 32, 32
_N = _B * _W * _H * _L
_D = 256
_NUM_EMB = (16, 8, 4, 6)
_OFFS = (0, 16, 24, 28)
_TBL_ROWS = 40
_SENT_ROW = 34

_COUNT = 13107
_VPAD = 13312
_SENT_SLOT = 13200
_VTOT = 4 * _VPAD

_ROWS, _COLS = 512, 256

_NC, _NS, _LANES = 2, 16, 16
_NW = _NC * _NS
_PER_TILE = (4 * _N) // _NW


def _rank_body(m_ref, rank_ref):
    iota_r = lax.broadcasted_iota(jnp.int32, (_COLS, _COLS), 0)
    iota_c = lax.broadcasted_iota(jnp.int32, (_COLS, _COLS), 1)
    upper = (iota_r <= iota_c).astype(jnp.float32)
    iota_r2 = lax.broadcasted_iota(jnp.int32, (_ROWS, _ROWS), 0)
    iota_c2 = lax.broadcasted_iota(jnp.int32, (_ROWS, _ROWS), 1)
    strict_lower = (iota_c2 < iota_r2).astype(jnp.float32)
    for i in range(4):
        m = m_ref[i]
        m_f = m.astype(jnp.float32)
        row_incl = lax.dot(m_f, upper, precision=lax.Precision.HIGHEST)
        row_tot = row_incl[:, _COLS - 1:_COLS]
        col_excl = lax.dot(strict_lower, row_tot,
                           precision=lax.Precision.HIGHEST)
        excl = row_incl - m_f + col_excl
        rank = excl.astype(jnp.int32)
        rank = jnp.where(m == 1, rank, _SENT_SLOT) + (i * _VPAD)
        rank_ref[i] = rank


def _ranks(masks_i32):
    return pl.pallas_call(
        _rank_body,
        out_shape=jax.ShapeDtypeStruct((4, _ROWS, _COLS), jnp.int32),
    )(masks_i32)


def _sc_compiler_params():
    cp = pltpu.CompilerParams()
    if "needs_layout_passes" in pltpu.CompilerParams.__dataclass_fields__:
        cp = dataclasses.replace(cp, needs_layout_passes=False)
    return cp


def _gather_body(vals_hbm, idx_hbm, out_hbm, vals_v, idx_v, out_v, sem):
    wid = lax.axis_index("s") * _NC + lax.axis_index("c")
    base = wid * _PER_TILE
    pltpu.async_copy(vals_hbm, vals_v, sem).wait()
    pltpu.async_copy(idx_hbm.at[pl.ds(base, _PER_TILE)], idx_v, sem).wait()

    @pl.loop(0, _PER_TILE, step=_LANES)
    def _(i):
        idxv = idx_v[pl.ds(i, _LANES)]
        out_v[pl.ds(i, _LANES)] = plsc.load_gather(vals_v, [idxv])

    pltpu.async_copy(out_v, out_hbm.at[pl.ds(base, _PER_TILE)], sem).wait()


def _sc_gather(vals_all, rank_flat):
    mesh = plsc.VectorSubcoreMesh(core_axis_name="c", subcore_axis_name="s")
    k = pl.kernel(
        _gather_body,
        out_type=jax.ShapeDtypeStruct((4 * _N,), jnp.int32),
        mesh=mesh,
        scratch_types=[
            pltpu.VMEM((_VTOT,), jnp.int32),
            pltpu.VMEM((_PER_TILE,), jnp.int32),
            pltpu.VMEM((_PER_TILE,), jnp.int32),
            pltpu.SemaphoreType.DMA,
        ],
        compiler_params=_sc_compiler_params(),
    )
    return k(vals_all, rank_flat)


_CR = 8


def _expand_body(sel_ref, tbl_ref, out_ref):
    iota40 = lax.broadcasted_iota(jnp.int32, (_TBL_ROWS, _COLS), 0)
    tbl = tbl_ref[...]
    for r in range(_CR):
        st = None
        for i in range(4):
            sel = (sel_ref[i, r, :] + _OFFS[i]).reshape(1, _COLS)
            eq = jnp.broadcast_to(sel, (_TBL_ROWS, _COLS)) == iota40
            st = eq.astype(jnp.int32) if st is None else st + eq
        tile = lax.dot_general(st.astype(jnp.float32), tbl,
                               (((0,), (0,)), ((), ())),
                               preferred_element_type=jnp.float32)
        out_ref[pl.ds(r * _COLS, _COLS), :] = tile


def _expand(sel, table40):
    return pl.pallas_call(
        _expand_body,
        grid=(_ROWS // _CR,),
        in_specs=[
            pl.BlockSpec((4, _CR, _COLS), lambda j: (0, j, 0)),
            pl.BlockSpec((_TBL_ROWS, _D), lambda j: (0, 0)),
        ],
        out_specs=pl.BlockSpec((_CR * _COLS, _D), lambda j: (j, 0)),
        out_shape=jax.ShapeDtypeStruct((_N, _D), jnp.float32),
        compiler_params=pltpu.CompilerParams(
            dimension_semantics=("parallel",),
        ),
    )(sel, table40)


def kernel(block_type_grid, mask_0, mask_1, mask_2, mask_3,
           values_0, values_1, values_2, values_3,
           table_0, table_1, table_2, table_3):
    masks = jnp.stack([m.reshape(_ROWS, _COLS)
                       for m in (mask_0, mask_1, mask_2, mask_3)])
    masks_i32 = masks.astype(jnp.int32)

    vals = []
    for v, off in zip((values_0, values_1, values_2, values_3), _OFFS):
        vals.append(jnp.pad(v, (0, _VPAD - v.shape[0]),
                            constant_values=_SENT_ROW - off))
    vals_all = jnp.concatenate(vals)

    table40 = jnp.concatenate(
        [table_0, table_1, table_2, table_3,
         jnp.zeros((_TBL_ROWS - sum(_NUM_EMB), _D), jnp.float32)])

    rank = _ranks(masks_i32)
    sel = _sc_gather(vals_all, rank.reshape(4 * _N))
    out = _expand(sel.reshape(4, _ROWS, _COLS), table40)
    return out.reshape(_B, _W, _H, _L, _D)


# speedup vs baseline: 1.1055x; 1.1055x over previous
import dataclasses
import functools

import jax
import jax.numpy as jnp
from jax import lax
from jax.experimental import pallas as pl
from jax.experimental.pallas import tpu as pltpu
from jax.experimental.pallas import tpu_sc as plsc

_B, _W, _H, _L = 4, 32, 32, 32
_N = _B * _W * _H * _L
_D = 256
_NUM_EMB = (16, 8, 4, 6)
_OFFS = (0, 16, 24, 28)
_TBL_ROWS = 40
_SENT_ROW = 34

_COUNT = 13107
_VPAD = 13312
_SENT_SLOT = 13200
_VTOT = 4 * _VPAD

_ROWS, _COLS = 512, 256

_NC, _NS, _LANES = 2, 16, 16
_NW = _NC * _NS
_PER_TILE = (4 * _N) // _NW


def _rank_body(m_ref, rank_ref):
    iota_r = lax.broadcasted_iota(jnp.int32, (_COLS, _COLS), 0)
    iota_c = lax.broadcasted_iota(jnp.int32, (_COLS, _COLS), 1)
    upper = (iota_r <= iota_c).astype(jnp.float32)
    iota_r2 = lax.broadcasted_iota(jnp.int32, (_ROWS, _ROWS), 0)
    iota_c2 = lax.broadcasted_iota(jnp.int32, (_ROWS, _ROWS), 1)
    strict_lower = (iota_c2 < iota_r2).astype(jnp.float32)
    for i in range(4):
        m = m_ref[i]
        m_f = m.astype(jnp.float32)
        row_incl = lax.dot(m_f, upper, precision=lax.Precision.HIGHEST)
        row_tot = row_incl[:, _COLS - 1:_COLS]
        col_excl = lax.dot(strict_lower, row_tot,
                           precision=lax.Precision.HIGHEST)
        excl = row_incl - m_f + col_excl
        rank = excl.astype(jnp.int32)
        rank_ref[i] = jnp.where(m == 1, rank, _SENT_SLOT)


def _ranks(masks_i32):
    return pl.pallas_call(
        _rank_body,
        out_shape=jax.ShapeDtypeStruct((4, _ROWS, _COLS), jnp.int32),
    )(masks_i32)


def _sc_compiler_params():
    cp = pltpu.CompilerParams()
    if "needs_layout_passes" in pltpu.CompilerParams.__dataclass_fields__:
        cp = dataclasses.replace(cp, needs_layout_passes=False)
    return cp


def _gather_body(vals_hbm, idx_hbm, out_hbm, vals_v, idx_v, out_v, sem, sem2):
    wid = lax.axis_index("s") * _NC + lax.axis_index("c")
    head = wid // (_NW // 4)
    base = wid * _PER_TILE
    c1 = pltpu.async_copy(vals_hbm.at[pl.ds(head * _VPAD, _VPAD)], vals_v, sem)
    c2 = pltpu.async_copy(idx_hbm.at[pl.ds(base, _PER_TILE)], idx_v, sem2)
    c1.wait()
    c2.wait()

    @plsc.parallel_loop(0, _PER_TILE, _LANES, unroll=8)
    def _(i):
        idxv = idx_v[pl.ds(i, _LANES)]
        out_v[pl.ds(i, _LANES)] = plsc.load_gather(vals_v, [idxv])

    pltpu.async_copy(out_v, out_hbm.at[pl.ds(base, _PER_TILE)], sem).wait()


def _sc_gather(vals_all, rank_flat):
    mesh = plsc.VectorSubcoreMesh(core_axis_name="c", subcore_axis_name="s")
    k = pl.kernel(
        _gather_body,
        out_type=jax.ShapeDtypeStruct((4 * _N,), jnp.int32),
        mesh=mesh,
        scratch_types=[
            pltpu.VMEM((_VPAD,), jnp.int32),
            pltpu.VMEM((_PER_TILE,), jnp.int32),
            pltpu.VMEM((_PER_TILE,), jnp.int32),
            pltpu.SemaphoreType.DMA,
            pltpu.SemaphoreType.DMA,
        ],
        compiler_params=_sc_compiler_params(),
    )
    return k(vals_all, rank_flat)


_CR = 8


def _expand_body(sel_ref, tbl_ref, out_ref):
    iota40 = lax.broadcasted_iota(jnp.int32, (_TBL_ROWS, _COLS), 0)
    tbl = tbl_ref[...]
    for r in range(_CR):
        st = None
        for i in range(4):
            sel = (sel_ref[i, r, :] + _OFFS[i]).reshape(1, _COLS)
            eq = jnp.broadcast_to(sel, (_TBL_ROWS, _COLS)) == iota40
            st = eq.astype(jnp.int32) if st is None else st + eq
        tile = lax.dot_general(st.astype(jnp.float32), tbl,
                               (((0,), (0,)), ((), ())),
                               preferred_element_type=jnp.float32)
        out_ref[pl.ds(r * _COLS, _COLS), :] = tile


def _expand(sel, table40):
    return pl.pallas_call(
        _expand_body,
        grid=(_ROWS // _CR,),
        in_specs=[
            pl.BlockSpec((4, _CR, _COLS), lambda j: (0, j, 0)),
            pl.BlockSpec((_TBL_ROWS, _D), lambda j: (0, 0)),
        ],
        out_specs=pl.BlockSpec((_CR * _COLS, _D), lambda j: (j, 0)),
        out_shape=jax.ShapeDtypeStruct((_N, _D), jnp.float32),
        compiler_params=pltpu.CompilerParams(
            dimension_semantics=("parallel",),
        ),
    )(sel, table40)


def kernel(block_type_grid, mask_0, mask_1, mask_2, mask_3,
           values_0, values_1, values_2, values_3,
           table_0, table_1, table_2, table_3):
    masks = jnp.stack([m.reshape(_ROWS, _COLS)
                       for m in (mask_0, mask_1, mask_2, mask_3)])
    masks_i32 = masks.astype(jnp.int32)

    vals = []
    for v, off in zip((values_0, values_1, values_2, values_3), _OFFS):
        vals.append(jnp.pad(v, (0, _VPAD - v.shape[0]),
                            constant_values=_SENT_ROW - off))
    vals_all = jnp.concatenate(vals)

    table40 = jnp.concatenate(
        [table_0, table_1, table_2, table_3,
         jnp.zeros((_TBL_ROWS - sum(_NUM_EMB), _D), jnp.float32)])

    rank = _ranks(masks_i32)
    sel = _sc_gather(vals_all, rank.reshape(4 * _N))
    out = _expand(sel.reshape(4, _ROWS, _COLS), table40)
    return out.reshape(_B, _W, _H, _L, _D)


# speedup vs baseline: 1.3151x; 1.1895x over previous
import dataclasses
import functools

import jax
import jax.numpy as jnp
from jax import lax
from jax.experimental import pallas as pl
from jax.experimental.pallas import tpu as pltpu
from jax.experimental.pallas import tpu_sc as plsc

_B, _W, _H, _L = 4, 32, 32, 32
_N = _B * _W * _H * _L
_D = 256
_NUM_EMB = (16, 8, 4, 6)
_OFFS = (0, 16, 24, 28)
_TBL_ROWS = 40
_SENT_ROW = 34

_COUNT = 13107
_VPAD = 13312
_SENT_SLOT = 13200
_VTOT = 4 * _VPAD

_ROWS, _COLS = 512, 256

_NC, _NS, _LANES = 2, 16, 16
_NW = _NC * _NS
_PER_TILE = (4 * _N) // _NW


def _rank_body(m_ref, rank_ref):
    iota_r = lax.broadcasted_iota(jnp.int32, (_COLS, _COLS), 0)
    iota_c = lax.broadcasted_iota(jnp.int32, (_COLS, _COLS), 1)
    upper = (iota_r <= iota_c).astype(jnp.float32)
    iota_r2 = lax.broadcasted_iota(jnp.int32, (_ROWS, _ROWS), 0)
    iota_c2 = lax.broadcasted_iota(jnp.int32, (_ROWS, _ROWS), 1)
    strict_lower = (iota_c2 < iota_r2).astype(jnp.float32)
    for i in range(4):
        m = m_ref[i]
        m_f = m.astype(jnp.float32)
        row_incl = lax.dot(m_f, upper, precision=lax.Precision.HIGHEST)
        row_tot = row_incl[:, _COLS - 1:_COLS]
        col_excl = lax.dot(strict_lower, row_tot,
                           precision=lax.Precision.HIGHEST)
        excl = row_incl - m_f + col_excl
        rank = excl.astype(jnp.int32)
        rank_ref[i] = jnp.where(m == 1, rank, _SENT_SLOT)


def _ranks(masks_i32):
    return pl.pallas_call(
        _rank_body,
        out_shape=jax.ShapeDtypeStruct((4, _ROWS, _COLS), jnp.int32),
    )(masks_i32)


def _sc_compiler_params():
    cp = pltpu.CompilerParams()
    if "needs_layout_passes" in pltpu.CompilerParams.__dataclass_fields__:
        cp = dataclasses.replace(cp, needs_layout_passes=False)
    return cp


def _gather_body(vals_hbm, idx_hbm, out_hbm, vals_v, idx_v, out_v, sem, sem2):
    wid = lax.axis_index("s") * _NC + lax.axis_index("c")
    head = wid // (_NW // 4)
    base = wid * _PER_TILE
    c1 = pltpu.async_copy(vals_hbm.at[pl.ds(head * _VPAD, _VPAD)], vals_v, sem)
    c2 = pltpu.async_copy(idx_hbm.at[pl.ds(base, _PER_TILE)], idx_v, sem2)
    c1.wait()
    c2.wait()

    @plsc.parallel_loop(0, _PER_TILE, _LANES, unroll=8)
    def _(i):
        idxv = idx_v[pl.ds(i, _LANES)]
        out_v[pl.ds(i, _LANES)] = plsc.load_gather(vals_v, [idxv])

    pltpu.async_copy(out_v, out_hbm.at[pl.ds(base, _PER_TILE)], sem).wait()


def _sc_gather(vals_all, rank_flat):
    mesh = plsc.VectorSubcoreMesh(core_axis_name="c", subcore_axis_name="s")
    k = pl.kernel(
        _gather_body,
        out_type=jax.ShapeDtypeStruct((4 * _N,), jnp.int32),
        mesh=mesh,
        scratch_types=[
            pltpu.VMEM((_VPAD,), jnp.int32),
            pltpu.VMEM((_PER_TILE,), jnp.int32),
            pltpu.VMEM((_PER_TILE,), jnp.int32),
            pltpu.SemaphoreType.DMA,
            pltpu.SemaphoreType.DMA,
        ],
        compiler_params=_sc_compiler_params(),
    )
    return k(vals_all, rank_flat)


_CR = 16


def _expand_body(sel_ref, tbl_ref, out_ref):
    iota40 = lax.broadcasted_iota(jnp.int32, (_TBL_ROWS, _COLS), 0)
    tbl = tbl_ref[...]
    for r in range(_CR):
        st = None
        for i in range(4):
            sel = (sel_ref[i, r, :] + _OFFS[i]).reshape(1, _COLS)
            eq = jnp.broadcast_to(sel, (_TBL_ROWS, _COLS)) == iota40
            st = eq.astype(jnp.int32) if st is None else st + eq
        tile = lax.dot_general(st.astype(jnp.float32), tbl,
                               (((0,), (0,)), ((), ())),
                               preferred_element_type=jnp.float32)
        out_ref[pl.ds(r * _COLS, _COLS), :] = tile


def _expand(sel, table40):
    return pl.pallas_call(
        _expand_body,
        grid=(_ROWS // _CR,),
        in_specs=[
            pl.BlockSpec((4, _CR, _COLS), lambda j: (0, j, 0)),
            pl.BlockSpec((_TBL_ROWS, _D), lambda j: (0, 0)),
        ],
        out_specs=pl.BlockSpec((_CR * _COLS, _D), lambda j: (j, 0)),
        out_shape=jax.ShapeDtypeStruct((_N, _D), jnp.float32),
        compiler_params=pltpu.CompilerParams(
            dimension_semantics=("parallel",),
        ),
    )(sel, table40)


def kernel(block_type_grid, mask_0, mask_1, mask_2, mask_3,
           values_0, values_1, values_2, values_3,
           table_0, table_1, table_2, table_3):
    masks = jnp.stack([m.reshape(_ROWS, _COLS)
                       for m in (mask_0, mask_1, mask_2, mask_3)])
    masks_i32 = masks.astype(jnp.int32)

    vals = []
    for v, off in zip((values_0, values_1, values_2, values_3), _OFFS):
        vals.append(jnp.pad(v, (0, _VPAD - v.shape[0]),
                            constant_values=_SENT_ROW - off))
    vals_all = jnp.concatenate(vals)

    table40 = jnp.concatenate(
        [table_0, table_1, table_2, table_3,
         jnp.zeros((_TBL_ROWS - sum(_NUM_EMB), _D), jnp.float32)])

    rank = _ranks(masks_i32)
    sel = _sc_gather(vals_all, rank.reshape(4 * _N))
    out = _expand(sel.reshape(4, _ROWS, _COLS), table40)
    return out.reshape(_B, _W, _H, _L, _D)


# speedup vs baseline: 1.3891x; 1.0563x over previous
import dataclasses
import functools

import jax
import jax.numpy as jnp
from jax import lax
from jax.experimental import pallas as pl
from jax.experimental.pallas import tpu as pltpu
from jax.experimental.pallas import tpu_sc as plsc

_B, _W, _H, _L = 4, 32, 32, 32
_N = _B * _W * _H * _L
_D = 256
_NUM_EMB = (16, 8, 4, 6)
_OFFS = (0, 16, 24, 28)
_TBL_ROWS = 40
_SENT_ROW = 34

_COUNT = 13107
_VPAD = 13312
_SENT_SLOT = 13200
_VTOT = 4 * _VPAD

_ROWS, _COLS = 512, 256

_NC, _NS, _LANES = 2, 16, 16
_NW = _NC * _NS
_PER_TILE = (4 * _N) // _NW


def _rank_body(m_ref, rank_ref):
    iota_r = lax.broadcasted_iota(jnp.int32, (_COLS, _COLS), 0)
    iota_c = lax.broadcasted_iota(jnp.int32, (_COLS, _COLS), 1)
    upper = (iota_r <= iota_c).astype(jnp.float32)
    iota_r2 = lax.broadcasted_iota(jnp.int32, (_ROWS, _ROWS), 0)
    iota_c2 = lax.broadcasted_iota(jnp.int32, (_ROWS, _ROWS), 1)
    strict_lower = (iota_c2 < iota_r2).astype(jnp.float32)
    for i in range(4):
        m = m_ref[i]
        m_f = m.astype(jnp.float32)
        row_incl = lax.dot(m_f, upper, precision=lax.Precision.HIGHEST)
        row_tot = row_incl[:, _COLS - 1:_COLS]
        col_excl = lax.dot(strict_lower, row_tot,
                           precision=lax.Precision.HIGHEST)
        excl = row_incl - m_f + col_excl
        rank = excl.astype(jnp.int32)
        rank_ref[i] = jnp.where(m == 1, rank, _SENT_SLOT)


def _ranks(masks_i32):
    return pl.pallas_call(
        _rank_body,
        out_shape=jax.ShapeDtypeStruct((4, _ROWS, _COLS), jnp.int32),
    )(masks_i32)


def _sc_compiler_params():
    cp = pltpu.CompilerParams()
    if "needs_layout_passes" in pltpu.CompilerParams.__dataclass_fields__:
        cp = dataclasses.replace(cp, needs_layout_passes=False)
    return cp


def _gather_body(vals_hbm, idx_hbm, out_hbm, vals_v, idx_v, out_v, sem, sem2):
    wid = lax.axis_index("s") * _NC + lax.axis_index("c")
    head = wid // (_NW // 4)
    base = wid * _PER_TILE
    c1 = pltpu.async_copy(vals_hbm.at[pl.ds(head * _VPAD, _VPAD)], vals_v, sem)
    c2 = pltpu.async_copy(idx_hbm.at[pl.ds(base, _PER_TILE)], idx_v, sem2)
    c1.wait()
    c2.wait()

    @plsc.parallel_loop(0, _PER_TILE, _LANES, unroll=8)
    def _(i):
        idxv = idx_v[pl.ds(i, _LANES)]
        out_v[pl.ds(i, _LANES)] = plsc.load_gather(vals_v, [idxv])

    pltpu.async_copy(out_v, out_hbm.at[pl.ds(base, _PER_TILE)], sem).wait()


def _sc_gather(vals_all, rank_flat):
    mesh = plsc.VectorSubcoreMesh(core_axis_name="c", subcore_axis_name="s")
    k = pl.kernel(
        _gather_body,
        out_type=jax.ShapeDtypeStruct((4 * _N,), jnp.int32),
        mesh=mesh,
        scratch_types=[
            pltpu.VMEM((_VPAD,), jnp.int32),
            pltpu.VMEM((_PER_TILE,), jnp.int32),
            pltpu.VMEM((_PER_TILE,), jnp.int32),
            pltpu.SemaphoreType.DMA,
            pltpu.SemaphoreType.DMA,
        ],
        compiler_params=_sc_compiler_params(),
    )
    return k(vals_all, rank_flat)


_CR = 32


def _expand_body(sel_ref, tbl_ref, out_ref):
    iota40 = lax.broadcasted_iota(jnp.int32, (_TBL_ROWS, _COLS), 0)
    tbl = tbl_ref[...]
    for r in range(_CR):
        st = None
        for i in range(4):
            sel = (sel_ref[i, r, :] + _OFFS[i]).reshape(1, _COLS)
            eq = jnp.broadcast_to(sel, (_TBL_ROWS, _COLS)) == iota40
            st = eq.astype(jnp.int32) if st is None else st + eq
        tile = lax.dot_general(st.astype(jnp.float32), tbl,
                               (((0,), (0,)), ((), ())),
                               preferred_element_type=jnp.float32)
        out_ref[pl.ds(r * _COLS, _COLS), :] = tile


def _expand(sel, table40):
    return pl.pallas_call(
        _expand_body,
        grid=(_ROWS // _CR,),
        in_specs=[
            pl.BlockSpec((4, _CR, _COLS), lambda j: (0, j, 0)),
            pl.BlockSpec((_TBL_ROWS, _D), lambda j: (0, 0)),
        ],
        out_specs=pl.BlockSpec((_CR * _COLS, _D), lambda j: (j, 0)),
        out_shape=jax.ShapeDtypeStruct((_N, _D), jnp.float32),
        compiler_params=pltpu.CompilerParams(
            dimension_semantics=("parallel",),
        ),
    )(sel, table40)


def kernel(block_type_grid, mask_0, mask_1, mask_2, mask_3,
           values_0, values_1, values_2, values_3,
           table_0, table_1, table_2, table_3):
    masks = jnp.stack([m.reshape(_ROWS, _COLS)
                       for m in (mask_0, mask_1, mask_2, mask_3)])
    masks_i32 = masks.astype(jnp.int32)

    vals = []
    for v, off in zip((values_0, values_1, values_2, values_3), _OFFS):
        vals.append(jnp.pad(v, (0, _VPAD - v.shape[0]),
                            constant_values=_SENT_ROW - off))
    vals_all = jnp.concatenate(vals)

    table40 = jnp.concatenate(
        [table_0, table_1, table_2, table_3,
         jnp.zeros((_TBL_ROWS - sum(_NUM_EMB), _D), jnp.float32)])

    rank = _ranks(masks_i32)
    sel = _sc_gather(vals_all, rank.reshape(4 * _N))
    out = _expand(sel.reshape(4, _ROWS, _COLS), table40)
    return out.reshape(_B, _W, _H, _L, _D)
